# baseline (speedup 1.0000x reference)
import jax
import jax.numpy as jnp
from jax import lax
from jax.experimental import pallas as pl
from jax.experimental.pallas import tpu as pltpu
from jax.experimental.pallas import tpu_sc as plsc

_B = 1024
_IN_DIM = 10000
_K = 16
_LANES = 16
_NTILE = 16
_ROWS_PER_T = _B // _NTILE
_F = 128
_TABW = (2 * _K + 1) * _F

_PREVS = (10000, 5120, 2560)
_D_PADS = (5120, 2560, 1280)
_RS = (8, 16, 32)
_HWORDS = 81920


def _lcn_stack(x, packed0, packed1, packed2):
    packs = (packed0, packed1, packed2)

    def body(x_hbm, tab0_hbm, tab1_hbm, tab2_hbm, h1_hbm, h2_hbm, h3_hbm,
             h_v, tab_a, tab_b, out_a, out_b,
             sem_h, sem_a, sem_b, sem_oa, sem_ob):
        net = lax.axis_index("c")
        tid = lax.axis_index("s")
        tabs = (tab0_hbm, tab1_hbm, tab2_hbm)
        hbms = (x_hbm, h1_hbm, h2_hbm, h3_hbm)

        for li in range(3):
            prev_w, d_pad, R = _PREVS[li], _D_PADS[li], _RS[li]
            h_hbm, out_hbm, tab_hbm = hbms[li], hbms[li + 1], tabs[li]
            nchunk = d_pad // _F
            P = nchunk // 2
            G = _ROWS_PER_T // R

            def compute_chunk(tab_v, out_v, R=R, prev_w=prev_w):
                @plsc.parallel_loop(0, _F // _LANES)
                def j_body(j):
                    j0 = j * _LANES
                    idxs = [tab_v[pl.ds(k * _F + j0, _LANES)]
                            for k in range(_K)]
                    bv = plsc.bitcast(
                        tab_v[pl.ds(2 * _K * _F + j0, _LANES)], jnp.float32)
                    for r0 in range(0, R, 8):
                        accs = [bv for _ in range(8)]
                        for k in range(_K):
                            wv = plsc.bitcast(
                                tab_v[pl.ds((_K + k) * _F + j0, _LANES)],
                                jnp.float32)
                            for r in range(8):
                                row = h_v.at[pl.ds((r0 + r) * prev_w, prev_w)]
                                gv = plsc.load_gather(row, [idxs[k]])
                                accs[r] = accs[r] + wv * gv
                        for r in range(8):
                            out_v[r0 + r, pl.ds(j0, _LANES)] = jnp.maximum(
                                accs[r], 0.0)

            def tab_dma(c, buf, sem, tab_hbm=tab_hbm):
                return pltpu.make_async_copy(tab_hbm.at[net, c], buf, sem)

            def out_dma(base, c, buf, sem, out_hbm=out_hbm, d_pad=d_pad, R=R):
                return pltpu.make_async_copy(
                    buf.at[pl.ds(0, R), :],
                    out_hbm.at[pl.ds(base, R), pl.ds(net * d_pad + c * _F, _F)],
                    sem)

            def group_body(g, _, R=R, prev_w=prev_w, P=P, nchunk=nchunk,
                           h_hbm=h_hbm, tab_dma=tab_dma, out_dma=out_dma,
                           compute_chunk=compute_chunk):
                base = tid * _ROWS_PER_T + g * R
                for r in range(R):
                    pltpu.async_copy(
                        h_hbm.at[base + r, pl.ds(net * prev_w, prev_w)],
                        h_v.at[pl.ds(r * prev_w, prev_w)], sem_h)
                tab_dma(0, tab_a, sem_a).start()
                for r in range(R):
                    pltpu.make_async_copy(
                        h_hbm.at[base + r, pl.ds(net * prev_w, prev_w)],
                        h_v.at[pl.ds(r * prev_w, prev_w)], sem_h).wait()

                def pair_body(p, _, P=P, base=base, tab_dma=tab_dma,
                              out_dma=out_dma, compute_chunk=compute_chunk):
                    c0 = 2 * p
                    tab_dma(c0 + 1, tab_b, sem_b).start()
                    tab_dma(c0, tab_a, sem_a).wait()

                    @pl.when(p > 0)
                    def _():
                        out_dma(base, c0 - 2, out_a, sem_oa).wait()

                    compute_chunk(tab_a, out_a)
                    out_dma(base, c0, out_a, sem_oa).start()

                    @pl.when(p < P - 1)
                    def _():
                        tab_dma(c0 + 2, tab_a, sem_a).start()

                    tab_dma(c0 + 1, tab_b, sem_b).wait()

                    @pl.when(p > 0)
                    def _():
                        out_dma(base, c0 - 1, out_b, sem_ob).wait()

                    compute_chunk(tab_b, out_b)
                    out_dma(base, c0 + 1, out_b, sem_ob).start()
                    return 0

                lax.fori_loop(0, P, pair_body, 0)
                out_dma(base, nchunk - 2, out_a, sem_oa).wait()
                out_dma(base, nchunk - 1, out_b, sem_ob).wait()
                return 0

            lax.fori_loop(0, G, group_body, 0)

    mesh = plsc.VectorSubcoreMesh(core_axis_name="c", subcore_axis_name="s")
    fn = pl.kernel(
        body,
        out_type=[jax.ShapeDtypeStruct((_B, 2 * _D_PADS[0]), jnp.float32),
                  jax.ShapeDtypeStruct((_B, 2 * _D_PADS[1]), jnp.float32),
                  jax.ShapeDtypeStruct((_B, 2 * _D_PADS[2]), jnp.float32)],
        mesh=mesh,
        compiler_params=pltpu.CompilerParams(use_tc_tiling_on_sc=False,
                                             needs_layout_passes=False),
        scratch_types=[
            pltpu.VMEM((_HWORDS,), jnp.float32),
            pltpu.VMEM((_TABW,), jnp.int32),
            pltpu.VMEM((_TABW,), jnp.int32),
            pltpu.VMEM((32, _F), jnp.float32),
            pltpu.VMEM((32, _F), jnp.float32),
            pltpu.SemaphoreType.DMA,
            pltpu.SemaphoreType.DMA,
            pltpu.SemaphoreType.DMA,
            pltpu.SemaphoreType.DMA,
            pltpu.SemaphoreType.DMA,
        ],
    )
    return fn(x, packed0, packed1, packed2)[2]


def _prep_tables(knn, wp, bp, wn, bn, d_pad):
    d = knn.shape[0]
    nchunk = d_pad // _F
    knnT = jnp.zeros((_K, d_pad), jnp.int32).at[:, :d].set(knn.T)

    def one(w, b):
        wT = jnp.zeros((_K, d_pad), jnp.float32).at[:, :d].set(w.T)
        bb = jnp.zeros((1, d_pad), jnp.float32).at[0, :d].set(b.reshape(-1))
        wi = lax.bitcast_convert_type(wT, jnp.int32)
        bi = lax.bitcast_convert_type(bb, jnp.int32)
        return jnp.concatenate([knnT, wi, bi], axis=0)

    packed = jnp.stack([one(wp, bp), one(wn, bn)], axis=0)
    packed = packed.reshape(2, 2 * _K + 1, nchunk, _F)
    packed = packed.transpose(0, 2, 1, 3).reshape(2, nchunk, _TABW)
    return packed


def kernel(x, knn0, w0p, b0p, w0n, b0n, knn1, w1p, b1p, w1n, b1n,
           knn2, w2p, b2p, w2n, b2n, fcw_p, fcb_p, fcw_n, fcb_n,
           fc3w, fc3b):
    packed0 = _prep_tables(knn0, w0p, b0p, w0n, b0n, _D_PADS[0])
    packed1 = _prep_tables(knn1, w1p, b1p, w1n, b1n, _D_PADS[1])
    packed2 = _prep_tables(knn2, w2p, b2p, w2n, b2n, _D_PADS[2])
    h = _lcn_stack(x, packed0, packed1, packed2)

    d2 = _D_PADS[2]
    big_fcw = jnp.zeros((4, 2 * d2), jnp.float32)
    big_fcw = big_fcw.at[0:2, 0:1250].set(fcw_p)
    big_fcw = big_fcw.at[2:4, d2:d2 + 1250].set(fcw_n)
    big_fcb = jnp.concatenate([fcb_p, fcb_n])
    hh = jnp.maximum(h @ big_fcw.T + big_fcb, 0.0)
    return hh @ fc3w.T + fc3b


# speedup vs baseline: 1.0855x; 1.0855x over previous
import jax
import jax.numpy as jnp
from jax import lax
from jax.experimental import pallas as pl
from jax.experimental.pallas import tpu as pltpu
from jax.experimental.pallas import tpu_sc as plsc

_B = 1024
_IN_DIM = 10000
_K = 16
_LANES = 16
_NTILE = 16
_ROWS_PER_T = _B // _NTILE
_NCHUNK = 10


def _lcn_layer(h, packed, d_pad, F, R):
    prev_w = h.shape[1] // 2
    R = int(R)
    G = _ROWS_PER_T // R
    P = _NCHUNK // 2

    def compute_chunk(tab_v, h_v, out_v):
        def j_body(j, _):
            j0 = j * _LANES
            bv = plsc.bitcast(tab_v[2 * _K, pl.ds(j0, _LANES)], jnp.float32)
            for r0 in range(0, R, 8):
                accs = [bv for _ in range(8)]
                for k in range(_K):
                    idx = tab_v[k, pl.ds(j0, _LANES)]
                    wv = plsc.bitcast(tab_v[_K + k, pl.ds(j0, _LANES)],
                                      jnp.float32)
                    for r in range(8):
                        gv = plsc.load_gather(h_v.at[r0 + r], [idx])
                        accs[r] = accs[r] + wv * gv
                for r in range(8):
                    out_v[r0 + r, pl.ds(j0, _LANES)] = jnp.maximum(accs[r], 0.0)
            return 0

        lax.fori_loop(0, F // _LANES, j_body, 0)

    def body(h_hbm, tab_hbm, out_hbm,
             h_v, tab_a, tab_b, out_a, out_b,
             sem_a, sem_b, sem_oa, sem_ob):
        net = lax.axis_index("c")
        tid = lax.axis_index("s")

        def tab_dma(c, buf, sem):
            return pltpu.make_async_copy(tab_hbm.at[net, :, c], buf, sem)

        def out_dma(base, c, buf, sem):
            return pltpu.make_async_copy(
                buf,
                out_hbm.at[pl.ds(base, R), pl.ds(net * d_pad + c * F, F)],
                sem)

        for g in range(G):
            base = tid * _ROWS_PER_T + g * R
            pltpu.sync_copy(
                h_hbm.at[pl.ds(base, R), pl.ds(net * prev_w, prev_w)], h_v)
            tab_dma(0, tab_a, sem_a).start()

            def pair_body(p, _):
                c0 = 2 * p
                tab_dma(c0 + 1, tab_b, sem_b).start()
                tab_dma(c0, tab_a, sem_a).wait()

                @pl.when(p > 0)
                def _():
                    out_dma(base, c0 - 2, out_a, sem_oa).wait()

                compute_chunk(tab_a, h_v, out_a)
                out_dma(base, c0, out_a, sem_oa).start()

                @pl.when(p < P - 1)
                def _():
                    tab_dma(c0 + 2, tab_a, sem_a).start()

                tab_dma(c0 + 1, tab_b, sem_b).wait()

                @pl.when(p > 0)
                def _():
                    out_dma(base, c0 - 1, out_b, sem_ob).wait()

                compute_chunk(tab_b, h_v, out_b)
                out_dma(base, c0 + 1, out_b, sem_ob).start()
                return 0

            lax.fori_loop(0, P, pair_body, 0)
            out_dma(base, _NCHUNK - 2, out_a, sem_oa).wait()
            out_dma(base, _NCHUNK - 1, out_b, sem_ob).wait()

    mesh = plsc.VectorSubcoreMesh(core_axis_name="c", subcore_axis_name="s")
    fn = pl.kernel(
        body,
        out_type=jax.ShapeDtypeStruct((_B, 2 * d_pad), jnp.float32),
        mesh=mesh,
        compiler_params=pltpu.CompilerParams(use_tc_tiling_on_sc=False,
                                             needs_layout_passes=False),
        scratch_types=[
            pltpu.VMEM((R, prev_w), jnp.float32),
            pltpu.VMEM((2 * _K + 1, F), jnp.int32),
            pltpu.VMEM((2 * _K + 1, F), jnp.int32),
            pltpu.VMEM((R, F), jnp.float32),
            pltpu.VMEM((R, F), jnp.float32),
            pltpu.SemaphoreType.DMA,
            pltpu.SemaphoreType.DMA,
            pltpu.SemaphoreType.DMA,
            pltpu.SemaphoreType.DMA,
        ],
    )
    return fn(h, packed)


def _prep_tables(knn, wp, bp, wn, bn, d_pad):
    d = knn.shape[0]
    F = d_pad // _NCHUNK
    knnT = jnp.zeros((_K, d_pad), jnp.int32).at[:, :d].set(knn.T)

    def one(w, b):
        wT = jnp.zeros((_K, d_pad), jnp.float32).at[:, :d].set(w.T)
        bb = jnp.zeros((1, d_pad), jnp.float32).at[0, :d].set(b.reshape(-1))
        wi = lax.bitcast_convert_type(wT, jnp.int32)
        bi = lax.bitcast_convert_type(bb, jnp.int32)
        return jnp.concatenate([knnT, wi, bi], axis=0)

    packed = jnp.stack([one(wp, bp), one(wn, bn)], axis=0)
    packed = packed.reshape(2, 2 * _K + 1, _NCHUNK, F)
    return packed, F


_D_PADS = (5120, 2560, 1280)
_RS = (8, 16, 32)


def kernel(x, knn0, w0p, b0p, w0n, b0n, knn1, w1p, b1p, w1n, b1n,
           knn2, w2p, b2p, w2n, b2n, fcw_p, fcb_p, fcw_n, fcb_n,
           fc3w, fc3b):
    layers = ((knn0, w0p, b0p, w0n, b0n),
              (knn1, w1p, b1p, w1n, b1n),
              (knn2, w2p, b2p, w2n, b2n))
    h = x
    for i in range(3):
        packed, F = _prep_tables(*layers[i], _D_PADS[i])
        h = _lcn_layer(h, packed, _D_PADS[i], F, _RS[i])
    d2 = _D_PADS[2]
    big_fcw = jnp.zeros((4, 2 * d2), jnp.float32)
    big_fcw = big_fcw.at[0:2, 0:1250].set(fcw_p)
    big_fcw = big_fcw.at[2:4, d2:d2 + 1250].set(fcw_n)
    big_fcb = jnp.concatenate([fcb_p, fcb_n])
    hh = jnp.maximum(h @ big_fcw.T + big_fcb, 0.0)
    return hh @ fc3w.T + fc3b


# speedup vs baseline: 1.1630x; 1.0714x over previous
import jax
import jax.numpy as jnp
from jax import lax
from jax.experimental import pallas as pl
from jax.experimental.pallas import tpu as pltpu
from jax.experimental.pallas import tpu_sc as plsc

_B = 1024
_IN_DIM = 10000
_K = 16
_LANES = 16
_NTILE = 16
_ROWS_PER_T = _B // _NTILE
_NCHUNK = 10


def _lcn_layer(h, packed, d_pad, F, R):
    prev_w = h.shape[1] // 2
    R = int(R)
    G = _ROWS_PER_T // R
    P = _NCHUNK // 2

    def compute_chunk(tab_v, h_v, out_v):
        def j_body(j, _):
            j0 = j * _LANES
            idxs = [tab_v[k, pl.ds(j0, _LANES)] for k in range(_K)]
            bv = plsc.bitcast(tab_v[2 * _K, pl.ds(j0, _LANES)], jnp.float32)
            for r0 in range(0, R, 8):
                accs = [bv for _ in range(8)]
                for k in range(_K):
                    wv = plsc.bitcast(tab_v[_K + k, pl.ds(j0, _LANES)],
                                      jnp.float32)
                    for r in range(8):
                        gv = plsc.load_gather(h_v.at[r0 + r], [idxs[k]])
                        accs[r] = accs[r] + wv * gv
                for r in range(8):
                    out_v[r0 + r, pl.ds(j0, _LANES)] = jnp.maximum(accs[r], 0.0)
            return 0

        lax.fori_loop(0, F // _LANES, j_body, 0)

    def body(h_hbm, tab_hbm, out_hbm,
             h_v, tab_a, tab_b, out_a, out_b,
             sem_a, sem_b, sem_oa, sem_ob):
        net = lax.axis_index("c")
        tid = lax.axis_index("s")

        def tab_dma(c, buf, sem):
            return pltpu.make_async_copy(tab_hbm.at[net, :, c], buf, sem)

        def out_dma(base, c, buf, sem):
            return pltpu.make_async_copy(
                buf,
                out_hbm.at[pl.ds(base, R), pl.ds(net * d_pad + c * F, F)],
                sem)

        for g in range(G):
            base = tid * _ROWS_PER_T + g * R
            pltpu.sync_copy(
                h_hbm.at[pl.ds(base, R), pl.ds(net * prev_w, prev_w)], h_v)
            tab_dma(0, tab_a, sem_a).start()

            def pair_body(p, _):
                c0 = 2 * p
                tab_dma(c0 + 1, tab_b, sem_b).start()
                tab_dma(c0, tab_a, sem_a).wait()

                @pl.when(p > 0)
                def _():
                    out_dma(base, c0 - 2, out_a, sem_oa).wait()

                compute_chunk(tab_a, h_v, out_a)
                out_dma(base, c0, out_a, sem_oa).start()

                @pl.when(p < P - 1)
                def _():
                    tab_dma(c0 + 2, tab_a, sem_a).start()

                tab_dma(c0 + 1, tab_b, sem_b).wait()

                @pl.when(p > 0)
                def _():
                    out_dma(base, c0 - 1, out_b, sem_ob).wait()

                compute_chunk(tab_b, h_v, out_b)
                out_dma(base, c0 + 1, out_b, sem_ob).start()
                return 0

            lax.fori_loop(0, P, pair_body, 0)
            out_dma(base, _NCHUNK - 2, out_a, sem_oa).wait()
            out_dma(base, _NCHUNK - 1, out_b, sem_ob).wait()

    mesh = plsc.VectorSubcoreMesh(core_axis_name="c", subcore_axis_name="s")
    fn = pl.kernel(
        body,
        out_type=jax.ShapeDtypeStruct((_B, 2 * d_pad), jnp.float32),
        mesh=mesh,
        compiler_params=pltpu.CompilerParams(use_tc_tiling_on_sc=False,
                                             needs_layout_passes=False),
        scratch_types=[
            pltpu.VMEM((R, prev_w), jnp.float32),
            pltpu.VMEM((2 * _K + 1, F), jnp.int32),
            pltpu.VMEM((2 * _K + 1, F), jnp.int32),
            pltpu.VMEM((R, F), jnp.float32),
            pltpu.VMEM((R, F), jnp.float32),
            pltpu.SemaphoreType.DMA,
            pltpu.SemaphoreType.DMA,
            pltpu.SemaphoreType.DMA,
            pltpu.SemaphoreType.DMA,
        ],
    )
    return fn(h, packed)


def _prep_tables(knn, wp, bp, wn, bn, d_pad):
    d = knn.shape[0]
    F = d_pad // _NCHUNK
    knnT = jnp.zeros((_K, d_pad), jnp.int32).at[:, :d].set(knn.T)

    def one(w, b):
        wT = jnp.zeros((_K, d_pad), jnp.float32).at[:, :d].set(w.T)
        bb = jnp.zeros((1, d_pad), jnp.float32).at[0, :d].set(b.reshape(-1))
        wi = lax.bitcast_convert_type(wT, jnp.int32)
        bi = lax.bitcast_convert_type(bb, jnp.int32)
        return jnp.concatenate([knnT, wi, bi], axis=0)

    packed = jnp.stack([one(wp, bp), one(wn, bn)], axis=0)
    packed = packed.reshape(2, 2 * _K + 1, _NCHUNK, F)
    return packed, F


_D_PADS = (5120, 2560, 1280)
_RS = (8, 16, 32)


def kernel(x, knn0, w0p, b0p, w0n, b0n, knn1, w1p, b1p, w1n, b1n,
           knn2, w2p, b2p, w2n, b2n, fcw_p, fcb_p, fcw_n, fcb_n,
           fc3w, fc3b):
    layers = ((knn0, w0p, b0p, w0n, b0n),
              (knn1, w1p, b1p, w1n, b1n),
              (knn2, w2p, b2p, w2n, b2n))
    h = x
    for i in range(3):
        packed, F = _prep_tables(*layers[i], _D_PADS[i])
        h = _lcn_layer(h, packed, _D_PADS[i], F, _RS[i])
    d2 = _D_PADS[2]
    big_fcw = jnp.zeros((4, 2 * d2), jnp.float32)
    big_fcw = big_fcw.at[0:2, 0:1250].set(fcw_p)
    big_fcw = big_fcw.at[2:4, d2:d2 + 1250].set(fcw_n)
    big_fcb = jnp.concatenate([fcb_p, fcb_n])
    hh = jnp.maximum(h @ big_fcw.T + big_fcb, 0.0)
    return hh @ fc3w.T + fc3b
